# baseline (speedup 1.0000x reference)
import jax
import jax.numpy as jnp
from jax import lax
from jax.experimental import pallas as pl
from jax.experimental.pallas import tpu as pltpu
from jax.experimental.pallas import tpu_sc as plsc

N_ROWS = 100000
N_UPD = 16384
N_COL = 128

NC = 2
NS = 16
L = 16
NW = NC * NS
COLS_PER_W = N_COL // NW

ROW_W = 128
ROWS_PER_BLK = 16
BLK = ROW_W * ROWS_PER_BLK
N_BLK = N_UPD // BLK
CHUNKS_PER_ROW = ROW_W // L

I_BITS = 14
I_MASK = (1 << I_BITS) - 1


def _tile_body(idx_hbm, upd_hbm, out_hbm,
               stamp, idxcol, ubuf, abuf, vbuf, tbuf, tbuff,
               sem_u, sem_sc):
    cid = lax.axis_index("c")
    sid = lax.axis_index("s")
    wid = sid * NC + cid
    lanes = lax.iota(jnp.int32, L)
    ones15 = jnp.full((L,), L - 1, jnp.int32)
    last_e = jnp.full((L,), N_UPD - 1, jnp.int32)

    for cc in range(COLS_PER_W):
        j = wid * COLS_PER_W + cc

        jbase = j * N_UPD

        pltpu.sync_copy(idx_hbm.at[pl.ds(jbase, N_UPD)], idxcol)

        def pass_a(k, _):
            base = k * L
            v = idxcol[pl.ds(base, L)]
            key = v * (1 << I_BITS) + (lanes + base)
            k_s = lax.sort(key, is_stable=False)
            v_s = lax.shift_right_logical(k_s, I_BITS)
            i_s = jnp.bitwise_and(k_s, I_MASK)
            tbuf[...] = v_s
            nxt = plsc.load_gather(tbuf, [jnp.minimum(lanes + 1, ones15)])
            is_last = jnp.logical_or(nxt != v_s, lanes == L - 1)
            plsc.store_scatter(stamp, [v_s], i_s, mask=is_last)
            return 0

        lax.fori_loop(0, N_UPD // L, pass_a, 0)

        v_last = plsc.load_gather(idxcol, [last_e])
        addr_last = v_last * N_COL + j
        pltpu.sync_copy(upd_hbm.at[pl.ds(jbase + N_UPD - L, L)], tbuff)
        u_last = plsc.load_gather(tbuff, [ones15])

        pltpu.async_copy(upd_hbm.at[pl.ds(jbase, BLK)], ubuf.at[0], sem_u)

        def pass_b(b, _):
            nb = lax.rem(b, 2)
            pltpu.make_async_copy(
                upd_hbm.at[pl.ds(jbase + b * BLK, BLK)], ubuf.at[nb],
                sem_u).wait()

            @pl.when(b < N_BLK - 1)
            def _():
                pltpu.async_copy(
                    upd_hbm.at[pl.ds(jbase + (b + 1) * BLK, BLK)],
                    ubuf.at[lax.rem(b + 1, 2)], sem_u)

            nbuf = lax.rem(b, 2) * ROWS_PER_BLK

            def one_row(r, _):
                rb = nbuf + r
                for c8 in range(CHUNKS_PER_ROW):
                    base = b * BLK + r * ROW_W + c8 * L
                    v = idxcol[pl.ds(base, L)]
                    u = ubuf[nb, pl.ds(r * ROW_W + c8 * L, L)]
                    w = plsc.load_gather(stamp, [v])
                    m = w == (lanes + base)
                    addr = v * N_COL + j
                    ffs = jnp.minimum(plsc.all_reduce_ffs(m), ones15)
                    cnt = plsc.all_reduce_population_count(m)
                    f_addr = jnp.where(cnt > 0, jnp.take(addr, ffs), addr_last)
                    f_val = jnp.where(cnt > 0, jnp.take(u, ffs), u_last)
                    abuf[rb, pl.ds(c8 * L, L)] = jnp.where(m, addr, f_addr)
                    vbuf[rb, pl.ds(c8 * L, L)] = jnp.where(m, u, f_val)
                pltpu.async_copy(vbuf.at[rb], out_hbm.at[abuf.at[rb]], sem_sc)
                return 0

            lax.fori_loop(0, ROWS_PER_BLK, one_row, 0)

            @pl.when(b >= 1)
            def _():
                prev = lax.rem(b - 1, 2) * ROWS_PER_BLK

                def drain(r, _):
                    pltpu.make_async_copy(
                        vbuf.at[prev + r], out_hbm.at[abuf.at[prev + r]],
                        sem_sc).wait()
                    return 0

                lax.fori_loop(0, ROWS_PER_BLK, drain, 0)
            return 0

        lax.fori_loop(0, N_BLK, pass_b, 0)

        def drain_tail(r, _):
            last = lax.rem(N_BLK - 1, 2) * ROWS_PER_BLK
            pltpu.make_async_copy(
                vbuf.at[last + r], out_hbm.at[abuf.at[last + r]],
                sem_sc).wait()
            return 0

        lax.fori_loop(0, ROWS_PER_BLK, drain_tail, 0)


def _build_sc_call():
    mesh = plsc.VectorSubcoreMesh(
        core_axis_name="c", subcore_axis_name="s",
        num_cores=NC, num_subcores=NS)
    return pl.kernel(
        _tile_body,
        out_type=(),
        mesh=mesh,
        scratch_types=[
            pltpu.VMEM((N_ROWS,), jnp.int32),
            pltpu.VMEM((N_UPD,), jnp.int32),
            pltpu.VMEM((2, BLK), jnp.float32),
            pltpu.VMEM((2 * ROWS_PER_BLK, ROW_W), jnp.int32),
            pltpu.VMEM((2 * ROWS_PER_BLK, ROW_W), jnp.float32),
            pltpu.VMEM((L,), jnp.int32),
            pltpu.VMEM((L,), jnp.float32),
            pltpu.SemaphoreType.DMA,
            pltpu.SemaphoreType.DMA,
        ],
        compiler_params=pltpu.CompilerParams(needs_layout_passes=False),
        name="sc_scatter_overwrite",
    )


def kernel(data, indices, updates):
    idx_t = jnp.reshape(jnp.transpose(indices), (-1,))
    upd_t = jnp.reshape(jnp.transpose(updates), (-1,))
    out_ref = jax.new_ref(jnp.reshape(data, (-1,)))
    _build_sc_call()(idx_t, upd_t, out_ref)
    return jnp.reshape(out_ref[...], (N_ROWS, N_COL))


# speedup vs baseline: 3.2188x; 3.2188x over previous
import jax
import jax.numpy as jnp
from jax import lax
from jax.experimental import pallas as pl
from jax.experimental.pallas import tpu as pltpu
from jax.experimental.pallas import tpu_sc as plsc

N_ROWS = 100000
N_UPD = 16384
N_COL = 128

NC = 2
NS = 16
L = 16
NW = NC * NS
COLS_PER_W = N_COL // NW

I_BITS = 14
I_MASK = (1 << I_BITS) - 1

ROW_W = 128
BR = 16
BLK = BR * ROW_W
N_BLK_A = N_UPD // BLK
N_ROWS_PAD = 100352
N_OBLK = N_ROWS_PAD // BLK
UROWS = N_UPD // ROW_W


def _tile_body(idx_hbm, upd_hbm, dataT_hbm, outT_hbm,
               stamp, ucol, ibuf, dbuf, obuf, tbuf,
               sem_i, sem_d, sem_o0, sem_o1):
    cid = lax.axis_index("c")
    sid = lax.axis_index("s")
    wid = sid * NC + cid
    lanes = lax.iota(jnp.int32, L)
    ones15 = jnp.full((L,), L - 1, jnp.int32)
    neg1 = jnp.full((L,), -1, jnp.int32)
    zeros = jnp.zeros((L,), jnp.int32)

    for cc in range(COLS_PER_W):
        j = wid * COLS_PER_W + cc

        pltpu.async_copy(upd_hbm.at[j], ucol, sem_d)

        def init_stamp(w, _):
            stamp[pl.ds(w * L, L)] = neg1
            return 0

        lax.fori_loop(0, N_ROWS_PAD // L, init_stamp, 0)

        pltpu.async_copy(idx_hbm.at[j, pl.ds(0, BR)], ibuf.at[0], sem_i)

        def pass_a_blk(b, _):
            nb = lax.rem(b, 2)
            pltpu.make_async_copy(
                idx_hbm.at[j, pl.ds(b * BR, BR)], ibuf.at[nb], sem_i).wait()

            @pl.when(b < N_BLK_A - 1)
            def _():
                pltpu.async_copy(
                    idx_hbm.at[j, pl.ds((b + 1) * BR, BR)],
                    ibuf.at[lax.rem(b + 1, 2)], sem_i)

            def chunk_row(r, _):
                for c8 in range(ROW_W // L):
                    v = ibuf[nb, r, pl.ds(c8 * L, L)]
                    base = b * BLK + r * ROW_W + c8 * L
                    key = v * (1 << I_BITS) + (lanes + base)
                    k_s = lax.sort(key, is_stable=False)
                    v_s = lax.shift_right_logical(k_s, I_BITS)
                    i_s = jnp.bitwise_and(k_s, I_MASK)
                    tbuf[...] = v_s
                    nxt = plsc.load_gather(
                        tbuf, [jnp.minimum(lanes + 1, ones15)])
                    is_last = jnp.logical_or(nxt != v_s, lanes == L - 1)
                    plsc.store_scatter(stamp, [v_s], i_s, mask=is_last)
                return 0

            lax.fori_loop(0, BR, chunk_row, 0)
            return 0

        lax.fori_loop(0, N_BLK_A, pass_a_blk, 0)

        pltpu.make_async_copy(upd_hbm.at[j], ucol, sem_d).wait()

        def asm_block(nb, boff):
            def w_row(r, _):
                for c8 in range(ROW_W // L):
                    s = stamp[pl.ds(boff + r * ROW_W + c8 * L, L)]
                    m = s >= 0
                    ss = jnp.where(m, s, zeros)
                    uv = plsc.load_gather(
                        ucol,
                        [lax.shift_right_logical(ss, 7),
                         jnp.bitwise_and(ss, ROW_W - 1)])
                    dv = dbuf[nb, r, pl.ds(c8 * L, L)]
                    obuf[nb, r, pl.ds(c8 * L, L)] = jnp.where(m, uv, dv)
                return 0

            lax.fori_loop(0, BR, w_row, 0)

        pltpu.async_copy(dataT_hbm.at[j, pl.ds(0, BR)], dbuf.at[0], sem_d)

        def asm_pair(p, _):
            b0 = 2 * p
            b1 = 2 * p + 1
            pltpu.make_async_copy(
                dataT_hbm.at[j, pl.ds(b0 * BR, BR)], dbuf.at[0],
                sem_d).wait()
            pltpu.async_copy(
                dataT_hbm.at[j, pl.ds(b1 * BR, BR)], dbuf.at[1], sem_d)

            @pl.when(p >= 1)
            def _():
                pltpu.make_async_copy(
                    obuf.at[0],
                    outT_hbm.at[j, pl.ds((b0 - 2) * BR, BR)], sem_o0).wait()

            asm_block(0, b0 * BLK)
            pltpu.async_copy(
                obuf.at[0], outT_hbm.at[j, pl.ds(b0 * BR, BR)], sem_o0)

            pltpu.make_async_copy(
                dataT_hbm.at[j, pl.ds(b1 * BR, BR)], dbuf.at[1],
                sem_d).wait()
            pltpu.async_copy(
                dataT_hbm.at[j, pl.ds((b1 + 1) * BR, BR)], dbuf.at[0], sem_d)

            @pl.when(p >= 1)
            def _():
                pltpu.make_async_copy(
                    obuf.at[1],
                    outT_hbm.at[j, pl.ds((b1 - 2) * BR, BR)], sem_o1).wait()

            asm_block(1, b1 * BLK)
            pltpu.async_copy(
                obuf.at[1], outT_hbm.at[j, pl.ds(b1 * BR, BR)], sem_o1)
            return 0

        lax.fori_loop(0, (N_OBLK - 1) // 2, asm_pair, 0)

        fb = N_OBLK - 1
        pltpu.make_async_copy(
            dataT_hbm.at[j, pl.ds(fb * BR, BR)], dbuf.at[0], sem_d).wait()
        pltpu.make_async_copy(
            obuf.at[0], outT_hbm.at[j, pl.ds((fb - 2) * BR, BR)],
            sem_o0).wait()
        asm_block(0, fb * BLK)
        pltpu.async_copy(
            obuf.at[0], outT_hbm.at[j, pl.ds(fb * BR, BR)], sem_o0)
        pltpu.make_async_copy(
            obuf.at[1], outT_hbm.at[j, pl.ds((fb - 1) * BR, BR)],
            sem_o1).wait()
        pltpu.make_async_copy(
            obuf.at[0], outT_hbm.at[j, pl.ds(fb * BR, BR)], sem_o0).wait()


def _build_sc_call():
    mesh = plsc.VectorSubcoreMesh(
        core_axis_name="c", subcore_axis_name="s",
        num_cores=NC, num_subcores=NS)
    return pl.kernel(
        _tile_body,
        out_type=jax.ShapeDtypeStruct(
            (N_COL, N_ROWS_PAD // ROW_W, ROW_W), jnp.float32),
        mesh=mesh,
        scratch_types=[
            pltpu.VMEM((N_ROWS_PAD,), jnp.int32),
            pltpu.VMEM((UROWS, ROW_W), jnp.float32),
            pltpu.VMEM((2, BR, ROW_W), jnp.int32),
            pltpu.VMEM((2, BR, ROW_W), jnp.float32),
            pltpu.VMEM((2, BR, ROW_W), jnp.float32),
            pltpu.VMEM((L,), jnp.int32),
            pltpu.SemaphoreType.DMA,
            pltpu.SemaphoreType.DMA,
            pltpu.SemaphoreType.DMA,
            pltpu.SemaphoreType.DMA,
        ],
        compiler_params=pltpu.CompilerParams(needs_layout_passes=False),
        name="sc_scatter_overwrite",
    )


def kernel(data, indices, updates):
    idx_t = jnp.reshape(jnp.transpose(indices), (N_COL, UROWS, ROW_W))
    upd_t = jnp.reshape(jnp.transpose(updates), (N_COL, UROWS, ROW_W))
    data_t = jnp.reshape(
        jnp.pad(jnp.transpose(data), ((0, 0), (0, N_ROWS_PAD - N_ROWS))),
        (N_COL, N_ROWS_PAD // ROW_W, ROW_W))
    out_t = _build_sc_call()(idx_t, upd_t, data_t)
    out_t = jnp.reshape(out_t, (N_COL, N_ROWS_PAD))[:, :N_ROWS]
    return jnp.transpose(out_t)


# speedup vs baseline: 4.7211x; 1.4667x over previous
import jax
import jax.numpy as jnp
from jax import lax
from jax.experimental import pallas as pl
from jax.experimental.pallas import tpu as pltpu
from jax.experimental.pallas import tpu_sc as plsc

N_ROWS = 100000
N_UPD = 16384
N_COL = 128

NC = 2
NS = 16
L = 16
NW = NC * NS
COLS_PER_W = N_COL // NW

I_BITS = 14
I_MASK = (1 << I_BITS) - 1

ROW_W = 128
BR = 16
BLK = BR * ROW_W
N_BLK_A = N_UPD // BLK
N_ROWS_PAD = 100352
N_OBLK = N_ROWS_PAD // BLK
UROWS = N_UPD // ROW_W


def _tile_body(idx_hbm, upd_hbm, dataT_hbm, outT_hbm,
               stamp, ucol, ibuf, dbuf, obuf, tbuf,
               sem_i, sem_d, sem_o0, sem_o1):
    cid = lax.axis_index("c")
    sid = lax.axis_index("s")
    wid = sid * NC + cid
    lanes = lax.iota(jnp.int32, L)
    ones15 = jnp.full((L,), L - 1, jnp.int32)
    neg1 = jnp.full((L,), -1, jnp.int32)
    zeros = jnp.zeros((L,), jnp.int32)

    for cc in range(COLS_PER_W):
        j = wid * COLS_PER_W + cc

        pltpu.async_copy(upd_hbm.at[j], ucol, sem_d)

        @plsc.parallel_loop(0, N_ROWS_PAD // L, unroll=4)
        def init_stamp(w):
            stamp[pl.ds(w * L, L)] = neg1

        pltpu.async_copy(idx_hbm.at[j, pl.ds(0, BR)], ibuf.at[0], sem_i)

        def pass_a_blk(b, _):
            nb = lax.rem(b, 2)
            pltpu.make_async_copy(
                idx_hbm.at[j, pl.ds(b * BR, BR)], ibuf.at[nb], sem_i).wait()

            @pl.when(b < N_BLK_A - 1)
            def _():
                pltpu.async_copy(
                    idx_hbm.at[j, pl.ds((b + 1) * BR, BR)],
                    ibuf.at[lax.rem(b + 1, 2)], sem_i)

            def chunk_row(r, _):
                for c8 in range(ROW_W // L):
                    v = ibuf[nb, r, pl.ds(c8 * L, L)]
                    base = b * BLK + r * ROW_W + c8 * L
                    key = v * (1 << I_BITS) + (lanes + base)
                    k_s = lax.sort(key, is_stable=False)
                    v_s = lax.shift_right_logical(k_s, I_BITS)
                    i_s = jnp.bitwise_and(k_s, I_MASK)
                    tbuf[...] = v_s
                    nxt = plsc.load_gather(
                        tbuf, [jnp.minimum(lanes + 1, ones15)])
                    is_last = jnp.logical_or(nxt != v_s, lanes == L - 1)
                    plsc.store_scatter(stamp, [v_s], i_s, mask=is_last)
                return 0

            lax.fori_loop(0, BR, chunk_row, 0, unroll=2)
            return 0

        lax.fori_loop(0, N_BLK_A, pass_a_blk, 0)

        pltpu.make_async_copy(upd_hbm.at[j], ucol, sem_d).wait()

        def asm_block(nb, boff):
            @plsc.parallel_loop(0, BR, unroll=2)
            def w_row(r):
                for c8 in range(ROW_W // L):
                    s = stamp[pl.ds(boff + r * ROW_W + c8 * L, L)]
                    m = s >= 0
                    ss = jnp.where(m, s, zeros)
                    uv = plsc.load_gather(
                        ucol,
                        [lax.shift_right_logical(ss, 7),
                         jnp.bitwise_and(ss, ROW_W - 1)])
                    dv = dbuf[nb, r, pl.ds(c8 * L, L)]
                    obuf[nb, r, pl.ds(c8 * L, L)] = jnp.where(m, uv, dv)

        pltpu.async_copy(dataT_hbm.at[j, pl.ds(0, BR)], dbuf.at[0], sem_d)

        def asm_pair(p, _):
            b0 = 2 * p
            b1 = 2 * p + 1
            pltpu.make_async_copy(
                dataT_hbm.at[j, pl.ds(b0 * BR, BR)], dbuf.at[0],
                sem_d).wait()
            pltpu.async_copy(
                dataT_hbm.at[j, pl.ds(b1 * BR, BR)], dbuf.at[1], sem_d)

            @pl.when(p >= 1)
            def _():
                pltpu.make_async_copy(
                    obuf.at[0],
                    outT_hbm.at[j, pl.ds((b0 - 2) * BR, BR)], sem_o0).wait()

            asm_block(0, b0 * BLK)
            pltpu.async_copy(
                obuf.at[0], outT_hbm.at[j, pl.ds(b0 * BR, BR)], sem_o0)

            pltpu.make_async_copy(
                dataT_hbm.at[j, pl.ds(b1 * BR, BR)], dbuf.at[1],
                sem_d).wait()
            pltpu.async_copy(
                dataT_hbm.at[j, pl.ds((b1 + 1) * BR, BR)], dbuf.at[0], sem_d)

            @pl.when(p >= 1)
            def _():
                pltpu.make_async_copy(
                    obuf.at[1],
                    outT_hbm.at[j, pl.ds((b1 - 2) * BR, BR)], sem_o1).wait()

            asm_block(1, b1 * BLK)
            pltpu.async_copy(
                obuf.at[1], outT_hbm.at[j, pl.ds(b1 * BR, BR)], sem_o1)
            return 0

        lax.fori_loop(0, (N_OBLK - 1) // 2, asm_pair, 0)

        fb = N_OBLK - 1
        pltpu.make_async_copy(
            dataT_hbm.at[j, pl.ds(fb * BR, BR)], dbuf.at[0], sem_d).wait()
        pltpu.make_async_copy(
            obuf.at[0], outT_hbm.at[j, pl.ds((fb - 2) * BR, BR)],
            sem_o0).wait()
        asm_block(0, fb * BLK)
        pltpu.async_copy(
            obuf.at[0], outT_hbm.at[j, pl.ds(fb * BR, BR)], sem_o0)
        pltpu.make_async_copy(
            obuf.at[1], outT_hbm.at[j, pl.ds((fb - 1) * BR, BR)],
            sem_o1).wait()
        pltpu.make_async_copy(
            obuf.at[0], outT_hbm.at[j, pl.ds(fb * BR, BR)], sem_o0).wait()


def _build_sc_call():
    mesh = plsc.VectorSubcoreMesh(
        core_axis_name="c", subcore_axis_name="s",
        num_cores=NC, num_subcores=NS)
    return pl.kernel(
        _tile_body,
        out_type=jax.ShapeDtypeStruct(
            (N_COL, N_ROWS_PAD // ROW_W, ROW_W), jnp.float32),
        mesh=mesh,
        scratch_types=[
            pltpu.VMEM((N_ROWS_PAD,), jnp.int32),
            pltpu.VMEM((UROWS, ROW_W), jnp.float32),
            pltpu.VMEM((2, BR, ROW_W), jnp.int32),
            pltpu.VMEM((2, BR, ROW_W), jnp.float32),
            pltpu.VMEM((2, BR, ROW_W), jnp.float32),
            pltpu.VMEM((L,), jnp.int32),
            pltpu.SemaphoreType.DMA,
            pltpu.SemaphoreType.DMA,
            pltpu.SemaphoreType.DMA,
            pltpu.SemaphoreType.DMA,
        ],
        compiler_params=pltpu.CompilerParams(needs_layout_passes=False),
        name="sc_scatter_overwrite",
    )


def kernel(data, indices, updates):
    idx_t = jnp.reshape(jnp.transpose(indices), (N_COL, UROWS, ROW_W))
    upd_t = jnp.reshape(jnp.transpose(updates), (N_COL, UROWS, ROW_W))
    data_t = jnp.reshape(
        jnp.pad(jnp.transpose(data), ((0, 0), (0, N_ROWS_PAD - N_ROWS))),
        (N_COL, N_ROWS_PAD // ROW_W, ROW_W))
    out_t = _build_sc_call()(idx_t, upd_t, data_t)
    out_t = jnp.reshape(out_t, (N_COL, N_ROWS_PAD))[:, :N_ROWS]
    return jnp.transpose(out_t)


# speedup vs baseline: 4.8091x; 1.0186x over previous
import jax
import jax.numpy as jnp
from jax import lax
from jax.experimental import pallas as pl
from jax.experimental.pallas import tpu as pltpu
from jax.experimental.pallas import tpu_sc as plsc

N_ROWS = 100000
N_UPD = 16384
N_COL = 128

NC = 2
NS = 16
L = 16
NW = NC * NS
COLS_PER_W = N_COL // NW

I_BITS = 14
I_MASK = (1 << I_BITS) - 1

ROW_W = 128
BR = 16
BLK = BR * ROW_W
N_BLK_A = N_UPD // BLK
N_ROWS_PAD = 100352
N_OBLK = N_ROWS_PAD // BLK
UROWS = N_UPD // ROW_W


def _tile_body(idx_hbm, upd_hbm, dataT_hbm, outT_hbm,
               stamp, ucol, ibuf, dbuf, obuf, tbuf,
               sem_i, sem_d, sem_o0, sem_o1):
    cid = lax.axis_index("c")
    sid = lax.axis_index("s")
    wid = sid * NC + cid
    lanes = lax.iota(jnp.int32, L)
    ones15 = jnp.full((L,), L - 1, jnp.int32)
    neg1 = jnp.full((L,), -1, jnp.int32)
    zeros = jnp.zeros((L,), jnp.int32)

    for cc in range(COLS_PER_W):
        j = wid * COLS_PER_W + cc

        pltpu.async_copy(upd_hbm.at[j], ucol, sem_d)

        @plsc.parallel_loop(0, N_ROWS_PAD // L, unroll=4)
        def init_stamp(w):
            stamp[pl.ds(w * L, L)] = neg1

        pltpu.async_copy(idx_hbm.at[j, pl.ds(0, BR)], ibuf.at[0], sem_i)

        def pass_a_blk(b, _):
            nb = lax.rem(b, 2)
            pltpu.make_async_copy(
                idx_hbm.at[j, pl.ds(b * BR, BR)], ibuf.at[nb], sem_i).wait()

            @pl.when(b < N_BLK_A - 1)
            def _():
                pltpu.async_copy(
                    idx_hbm.at[j, pl.ds((b + 1) * BR, BR)],
                    ibuf.at[lax.rem(b + 1, 2)], sem_i)

            def chunk_row(r, _):
                for c8 in range(ROW_W // L):
                    v = ibuf[nb, r, pl.ds(c8 * L, L)]
                    base = b * BLK + r * ROW_W + c8 * L
                    key = v * (1 << I_BITS) + (lanes + base)
                    k_s = lax.sort(key, is_stable=False)
                    v_s = lax.shift_right_logical(k_s, I_BITS)
                    i_s = jnp.bitwise_and(k_s, I_MASK)
                    nxt = jnp.take(v_s, jnp.minimum(lanes + 1, ones15))
                    is_last = jnp.logical_or(nxt != v_s, lanes == L - 1)
                    plsc.store_scatter(stamp, [v_s], i_s, mask=is_last)
                return 0

            lax.fori_loop(0, BR, chunk_row, 0, unroll=2)
            return 0

        lax.fori_loop(0, N_BLK_A, pass_a_blk, 0)

        pltpu.make_async_copy(upd_hbm.at[j], ucol, sem_d).wait()

        def asm_block(nb, boff):
            @plsc.parallel_loop(0, BR, unroll=4)
            def w_row(r):
                for c8 in range(ROW_W // L):
                    s = stamp[pl.ds(boff + r * ROW_W + c8 * L, L)]
                    m = s >= 0
                    ss = jnp.where(m, s, zeros)
                    uv = plsc.load_gather(
                        ucol,
                        [lax.shift_right_logical(ss, 7),
                         jnp.bitwise_and(ss, ROW_W - 1)])
                    dv = dbuf[nb, r, pl.ds(c8 * L, L)]
                    obuf[nb, r, pl.ds(c8 * L, L)] = jnp.where(m, uv, dv)

        pltpu.async_copy(dataT_hbm.at[j, pl.ds(0, BR)], dbuf.at[0], sem_d)

        def asm_pair(p, _):
            b0 = 2 * p
            b1 = 2 * p + 1
            pltpu.make_async_copy(
                dataT_hbm.at[j, pl.ds(b0 * BR, BR)], dbuf.at[0],
                sem_d).wait()
            pltpu.async_copy(
                dataT_hbm.at[j, pl.ds(b1 * BR, BR)], dbuf.at[1], sem_d)

            @pl.when(p >= 1)
            def _():
                pltpu.make_async_copy(
                    obuf.at[0],
                    outT_hbm.at[j, pl.ds((b0 - 2) * BR, BR)], sem_o0).wait()

            asm_block(0, b0 * BLK)
            pltpu.async_copy(
                obuf.at[0], outT_hbm.at[j, pl.ds(b0 * BR, BR)], sem_o0)

            pltpu.make_async_copy(
                dataT_hbm.at[j, pl.ds(b1 * BR, BR)], dbuf.at[1],
                sem_d).wait()
            pltpu.async_copy(
                dataT_hbm.at[j, pl.ds((b1 + 1) * BR, BR)], dbuf.at[0], sem_d)

            @pl.when(p >= 1)
            def _():
                pltpu.make_async_copy(
                    obuf.at[1],
                    outT_hbm.at[j, pl.ds((b1 - 2) * BR, BR)], sem_o1).wait()

            asm_block(1, b1 * BLK)
            pltpu.async_copy(
                obuf.at[1], outT_hbm.at[j, pl.ds(b1 * BR, BR)], sem_o1)
            return 0

        lax.fori_loop(0, (N_OBLK - 1) // 2, asm_pair, 0)

        fb = N_OBLK - 1
        pltpu.make_async_copy(
            dataT_hbm.at[j, pl.ds(fb * BR, BR)], dbuf.at[0], sem_d).wait()
        pltpu.make_async_copy(
            obuf.at[0], outT_hbm.at[j, pl.ds((fb - 2) * BR, BR)],
            sem_o0).wait()
        asm_block(0, fb * BLK)
        pltpu.async_copy(
            obuf.at[0], outT_hbm.at[j, pl.ds(fb * BR, BR)], sem_o0)
        pltpu.make_async_copy(
            obuf.at[1], outT_hbm.at[j, pl.ds((fb - 1) * BR, BR)],
            sem_o1).wait()
        pltpu.make_async_copy(
            obuf.at[0], outT_hbm.at[j, pl.ds(fb * BR, BR)], sem_o0).wait()


def _build_sc_call():
    mesh = plsc.VectorSubcoreMesh(
        core_axis_name="c", subcore_axis_name="s",
        num_cores=NC, num_subcores=NS)
    return pl.kernel(
        _tile_body,
        out_type=jax.ShapeDtypeStruct(
            (N_COL, N_ROWS_PAD // ROW_W, ROW_W), jnp.float32),
        mesh=mesh,
        scratch_types=[
            pltpu.VMEM((N_ROWS_PAD,), jnp.int32),
            pltpu.VMEM((UROWS, ROW_W), jnp.float32),
            pltpu.VMEM((2, BR, ROW_W), jnp.int32),
            pltpu.VMEM((2, BR, ROW_W), jnp.float32),
            pltpu.VMEM((2, BR, ROW_W), jnp.float32),
            pltpu.VMEM((L,), jnp.int32),
            pltpu.SemaphoreType.DMA,
            pltpu.SemaphoreType.DMA,
            pltpu.SemaphoreType.DMA,
            pltpu.SemaphoreType.DMA,
        ],
        compiler_params=pltpu.CompilerParams(needs_layout_passes=False),
        name="sc_scatter_overwrite",
    )


def kernel(data, indices, updates):
    idx_t = jnp.reshape(jnp.transpose(indices), (N_COL, UROWS, ROW_W))
    upd_t = jnp.reshape(jnp.transpose(updates), (N_COL, UROWS, ROW_W))
    data_t = jnp.reshape(
        jnp.pad(jnp.transpose(data), ((0, 0), (0, N_ROWS_PAD - N_ROWS))),
        (N_COL, N_ROWS_PAD // ROW_W, ROW_W))
    out_t = _build_sc_call()(idx_t, upd_t, data_t)
    out_t = jnp.reshape(out_t, (N_COL, N_ROWS_PAD))[:, :N_ROWS]
    return jnp.transpose(out_t)
